# baseline (speedup 1.0000x reference)
import functools

import jax
import jax.numpy as jnp
from jax import lax
from jax.experimental import pallas as pl
from jax.experimental.pallas import tpu as pltpu
from jax.experimental.pallas import tpu_sc as plsc

_VOCAB = 1000000
_EMBED = 64
_BATCH = 16384
_NEG = 20
_COLS = _NEG + 1

_NC, _NS = 2, 16
_NW = _NC * _NS
_ROWS_PER_W = _BATCH // _NW
_R = 32
_NCHUNK = _ROWS_PER_W // _R
_CNROWS = _R * _COLS
_GLEN = 16
_GPC = _CNROWS // _GLEN
_CGP = 2
_CGL = _R // _CGP

_MESH = dict(core_axis_name="c", subcore_axis_name="s")
_CPARAMS = pltpu.CompilerParams(
    use_tc_tiling_on_sc=False, needs_layout_passes=False)


def _stage_a(cn_idx, output_table):
    @functools.partial(
        pl.kernel,
        out_type=jax.ShapeDtypeStruct((_NW, _NCHUNK, _CNROWS, _EMBED),
                                      jnp.float32),
        mesh=plsc.VectorSubcoreMesh(**_MESH),
        scratch_types=[
            pltpu.VMEM((_GPC, _GLEN), jnp.int32),
            pltpu.VMEM((_GPC, _GLEN), jnp.int32),
            pltpu.VMEM((_CNROWS, _EMBED), jnp.float32),
            pltpu.VMEM((_CNROWS, _EMBED), jnp.float32),
            pltpu.SemaphoreType.DMA,
            pltpu.SemaphoreType.DMA,
            pltpu.SemaphoreType.DMA,
            pltpu.SemaphoreType.DMA,
            pltpu.SemaphoreType.DMA,
            pltpu.SemaphoreType.DMA,
        ],
        compiler_params=_CPARAMS,
    )
    def gather_kernel(cnidx_hbm, otab_hbm, staged_hbm,
                      cnidx0, cnidx1, cnrows0, cnrows1,
                      isem0, isem1, gsem0, gsem1, osem0, osem1):
        wid = lax.axis_index("s") * _NC + lax.axis_index("c")
        cnidx = (cnidx0, cnidx1)
        cnrows = (cnrows0, cnrows1)
        isem = (isem0, isem1)
        gsem = (gsem0, gsem1)
        osem = (osem0, osem1)

        def idx_copy(c, p):
            return pltpu.make_async_copy(cnidx_hbm.at[wid, c], cnidx[p],
                                         isem[p])

        def fire_gathers(p):
            for j in range(_GPC):
                pltpu.make_async_copy(
                    otab_hbm.at[cnidx[p][j, :]],
                    cnrows[p].at[pl.ds(j * _GLEN, _GLEN)], gsem[p]).start()

        def drain_gathers(p):
            pltpu.make_async_copy(
                otab_hbm.at[pl.ds(0, _CNROWS)], cnrows[p], gsem[p]).wait()

        def out_copy(c, p):
            return pltpu.make_async_copy(cnrows[p], staged_hbm.at[wid, c],
                                         osem[p])

        idx_copy(0, 0).start()
        idx_copy(1, 1).start()
        out_copy(0, 0).start()
        out_copy(1, 1).start()
        idx_copy(0, 0).wait()
        out_copy(0, 0).wait()
        fire_gathers(0)
        idx_copy(2, 0).start()
        idx_copy(1, 1).wait()
        out_copy(1, 1).wait()
        fire_gathers(1)
        drain_gathers(0)
        out_copy(0, 0).start()
        idx_copy(3, 1).start()

        def pair_body(i, carry):
            c0 = 2 * i
            c1 = c0 + 1
            idx_copy(c0, 0).wait()
            out_copy(c0 - 2, 0).wait()
            fire_gathers(0)
            drain_gathers(1)
            out_copy(c1 - 2, 1).start()
            idx_copy((c0 + 2) & (_NCHUNK - 1), 0).start()
            idx_copy(c1, 1).wait()
            out_copy(c1 - 2, 1).wait()
            fire_gathers(1)
            drain_gathers(0)
            out_copy(c0, 0).start()
            idx_copy((c1 + 2) & (_NCHUNK - 1), 1).start()
            return carry

        lax.fori_loop(1, _NCHUNK // 2, pair_body, 0)

        drain_gathers(1)
        out_copy(_NCHUNK - 1, 1).start()
        idx_copy(0, 0).wait()
        idx_copy(1, 1).wait()
        out_copy(_NCHUNK - 2, 0).wait()
        out_copy(_NCHUNK - 1, 1).wait()

    return gather_kernel(cn_idx, output_table)


def _stage_b(center_idx, staged, input_table):
    @functools.partial(
        pl.kernel,
        out_type=jax.ShapeDtypeStruct((_NW, _NCHUNK, _COLS, _R), jnp.float32),
        mesh=plsc.VectorSubcoreMesh(**_MESH),
        scratch_types=[
            pltpu.VMEM((_CGP, _CGL), jnp.int32),
            pltpu.VMEM((_CGP, _CGL), jnp.int32),
            pltpu.VMEM((_R, _EMBED), jnp.float32),
            pltpu.VMEM((_R, _EMBED), jnp.float32),
            pltpu.VMEM((_CNROWS, _EMBED), jnp.float32),
            pltpu.VMEM((_CNROWS, _EMBED), jnp.float32),
            pltpu.VMEM((_COLS, _R), jnp.float32),
            pltpu.VMEM((_COLS, _R), jnp.float32),
            pltpu.SemaphoreType.DMA,
            pltpu.SemaphoreType.DMA,
            pltpu.SemaphoreType.DMA,
            pltpu.SemaphoreType.DMA,
            pltpu.SemaphoreType.DMA,
            pltpu.SemaphoreType.DMA,
        ],
        compiler_params=_CPARAMS,
    )
    def score_kernel(cidx_hbm, staged_hbm, itab_hbm, out_hbm,
                     cidx0, cidx1, crows0, crows1, cnrows0, cnrows1,
                     scores0, scores1,
                     isem0, isem1, gsem0, gsem1, osem0, osem1):
        wid = lax.axis_index("s") * _NC + lax.axis_index("c")
        lanes = lax.iota(jnp.int32, _NS)
        cidx = (cidx0, cidx1)
        crows = (crows0, crows1)
        cnrows = (cnrows0, cnrows1)
        scores = (scores0, scores1)
        isem = (isem0, isem1)
        gsem = (gsem0, gsem1)
        osem = (osem0, osem1)

        def idx_copy(c, p):
            return pltpu.make_async_copy(cidx_hbm.at[wid, c], cidx[p],
                                         isem[p])

        def fire_loads(c, p):
            for j in range(_CGP):
                pltpu.make_async_copy(
                    itab_hbm.at[cidx[p][j, :]],
                    crows[p].at[pl.ds(j * _CGL, _CGL)], gsem[p]).start()
            pltpu.make_async_copy(staged_hbm.at[wid, c], cnrows[p],
                                  gsem[p]).start()

        def drain_loads(p):
            pltpu.make_async_copy(
                itab_hbm.at[pl.ds(0, _R)], crows[p], gsem[p]).wait()
            pltpu.make_async_copy(
                itab_hbm.at[pl.ds(0, _CNROWS)], cnrows[p], gsem[p]).wait()

        def out_copy(c, p):
            return pltpu.make_async_copy(scores[p], out_hbm.at[wid, c],
                                         osem[p])

        def compute(p):
            for g in range(_R // _NS):
                r_vec = g * _NS + lanes
                cn_rows = [r_vec * _COLS + k for k in range(_COLS)]

                def d_body(d, accs):
                    d_vec = jnp.full((_NS,), d, jnp.int32)
                    cen = plsc.load_gather(crows[p], [r_vec, d_vec])
                    return tuple(
                        accs[k] + cen * plsc.load_gather(
                            cnrows[p], [cn_rows[k], d_vec])
                        for k in range(_COLS)
                    )

                accs = lax.fori_loop(
                    0, _EMBED, d_body,
                    tuple(jnp.zeros((_NS,), jnp.float32)
                          for _ in range(_COLS)))
                scores[p][0, pl.ds(g * _NS, _NS)] = -accs[0]
                for k in range(1, _COLS):
                    scores[p][k, pl.ds(g * _NS, _NS)] = accs[k]

        idx_copy(0, 0).start()
        idx_copy(0, 0).wait()
        fire_loads(0, 0)
        idx_copy(1, 1).start()
        out_copy(0, 0).start()
        out_copy(1, 1).start()

        def pair_body(i, carry):
            c0 = 2 * i
            c1 = c0 + 1
            c2 = (c0 + 2) & (_NCHUNK - 1)
            c3 = (c0 + 3) & (_NCHUNK - 1)

            idx_copy(c1, 1).wait()
            fire_loads(c1, 1)
            drain_loads(0)
            idx_copy(c2, 0).start()
            out_copy(c0, 0).wait()
            compute(0)
            out_copy(c0, 0).start()

            idx_copy(c2, 0).wait()
            fire_loads(c2, 0)
            drain_loads(1)
            idx_copy(c3, 1).start()
            out_copy(c1, 1).wait()
            compute(1)
            out_copy(c1, 1).start()
            return carry

        lax.fori_loop(0, _NCHUNK // 2, pair_body, 0)

        idx_copy(1, 1).wait()
        drain_loads(0)
        out_copy(_NCHUNK - 2, 0).wait()
        out_copy(_NCHUNK - 1, 1).wait()

    return score_kernel(center_idx, staged, input_table)


def _loss_body(s_ref, o_ref):
    t = s_ref[...]
    sp = jnp.maximum(t, 0.0) + jnp.log1p(jnp.exp(-jnp.abs(t)))
    o_ref[0, 0] = jnp.sum(sp) * (1.0 / _BATCH)


def kernel(center, context, negative, input_table, output_table):
    cn = jnp.concatenate([context[:, None], negative], axis=1)
    cn = cn.reshape(_NW, _NCHUNK, _GPC, _GLEN).astype(jnp.int32)
    cidx = center.reshape(_NW, _NCHUNK, _CGP, _CGL).astype(jnp.int32)

    staged = _stage_a(cn, output_table)
    scores = _stage_b(cidx, staged, input_table)

    flat = scores.reshape(_BATCH * _COLS // 128, 128)
    loss = pl.pallas_call(
        _loss_body,
        out_shape=jax.ShapeDtypeStruct((1, 1), jnp.float32),
        out_specs=pl.BlockSpec(memory_space=pltpu.SMEM),
    )(flat)
    return loss[0, 0]
